# baseline (speedup 1.0000x reference)
import jax
import jax.numpy as jnp
from jax import lax
from jax.experimental import pallas as pl
from jax.experimental.pallas import tpu as pltpu
from jax.experimental.pallas import tpu_sc as plsc

NUM_GRAPHS = 64
D = 512
N = 100000
NC = 2
NS = 16
NW = NC * NS
L = 16
TILE = 80
DJ = D // L
ACC_ROWS = NUM_GRAPHS * DJ


def _row_scale(xbuf, wbuf, r):
    p0 = jnp.zeros((L,), jnp.float32)
    p1 = jnp.zeros((L,), jnp.float32)
    p2 = jnp.zeros((L,), jnp.float32)
    p3 = jnp.zeros((L,), jnp.float32)
    for j in range(0, DJ, 4):
        p0 = p0 + xbuf[r, pl.ds(j * L, L)] * wbuf[pl.ds(j * L, L)]
        p1 = p1 + xbuf[r, pl.ds((j + 1) * L, L)] * wbuf[pl.ds((j + 1) * L, L)]
        p2 = p2 + xbuf[r, pl.ds((j + 2) * L, L)] * wbuf[pl.ds((j + 2) * L, L)]
        p3 = p3 + xbuf[r, pl.ds((j + 3) * L, L)] * wbuf[pl.ds((j + 3) * L, L)]
    att = (p0 + p1) + (p2 + p3)
    a = plsc.cumsum(att)[L - 1]
    av = jnp.full((L,), a, jnp.float32)
    return (1.0 / (1.0 + jnp.exp(-av)) + 1.0) * 0.5


def _make_sc_body(row_base, n_rows):
    nt = n_rows // TILE

    def _sc_body(x_hbm, b_hbm, watt_hbm, part_hbm, xbuf, bbuf, wbuf, acc):
        wid = lax.axis_index("s") * NC + lax.axis_index("c")
        base = nt // NW
        extra = nt % NW
        start = wid * base + jnp.minimum(wid, extra)
        count = base + (wid < extra).astype(jnp.int32)

        pltpu.sync_copy(watt_hbm.at[0], wbuf)

        def init_body(k, c):
            for j in range(DJ):
                acc[k, pl.ds(j * L, L)] = jnp.full((L,), -jnp.inf,
                                                   jnp.float32)
            return c

        lax.fori_loop(0, NUM_GRAPHS, init_body, 0)

        def tile_body(t, c):
            row0 = row_base + t * TILE
            pltpu.sync_copy(x_hbm.at[pl.ds(row0, TILE)], xbuf)
            pltpu.sync_copy(b_hbm.at[pl.ds(row0, TILE)],
                            bbuf.at[pl.ds(0, TILE)])
            seg_lo = bbuf[pl.ds(0, L)][0]
            seg_hi = bbuf[pl.ds(TILE - 1, L)][0]

            def uniform(_):
                def row_body(r, tm):
                    scale = _row_scale(xbuf, wbuf, r)
                    return tuple(
                        jnp.maximum(tm[j], xbuf[r, pl.ds(j * L, L)] * scale)
                        for j in range(DJ))

                tm0 = tuple(jnp.full((L,), -jnp.inf, jnp.float32)
                            for _ in range(DJ))
                tm = lax.fori_loop(0, TILE, row_body, tm0)
                for j in range(DJ):
                    acc[seg_lo, pl.ds(j * L, L)] = jnp.maximum(
                        acc[seg_lo, pl.ds(j * L, L)], tm[j])
                return 0

            def mixed(_):
                def row_body(r, c2):
                    scale = _row_scale(xbuf, wbuf, r)
                    seg = bbuf[pl.ds(r, L)][0]
                    for j in range(DJ):
                        yv = xbuf[r, pl.ds(j * L, L)] * scale
                        acc[seg, pl.ds(j * L, L)] = jnp.maximum(
                            acc[seg, pl.ds(j * L, L)], yv)
                    return c2

                lax.fori_loop(0, TILE, row_body, 0)
                return 0

            lax.cond(seg_lo == seg_hi, uniform, mixed, 0)
            return c

        lax.fori_loop(start, start + count, tile_body, 0)
        pltpu.sync_copy(acc, part_hbm.at[wid])

    return _sc_body


def _sc_partials(x, batch, W_att, row_base, n_rows):
    mesh = plsc.VectorSubcoreMesh(
        core_axis_name="c", subcore_axis_name="s",
        num_cores=NC, num_subcores=NS)
    f = pl.kernel(
        _make_sc_body(row_base, n_rows),
        out_type=jax.ShapeDtypeStruct((NW, NUM_GRAPHS, D), jnp.float32),
        mesh=mesh,
        compiler_params=pltpu.CompilerParams(
            needs_layout_passes=False, use_tc_tiling_on_sc=True),
        scratch_types=[
            pltpu.VMEM((TILE, D), jnp.float32),
            pltpu.VMEM((TILE + L,), jnp.int32),
            pltpu.VMEM((D,), jnp.float32),
            pltpu.VMEM((NUM_GRAPHS, D), jnp.float32),
        ],
    )
    return f(x, batch, W_att)


def _merge_body(part_ref, wout_ref, out_ref):
    def body(w, m):
        return jnp.maximum(m, part_ref[w])

    hg = lax.fori_loop(1, NW, body, part_ref[0])
    out_ref[...] = jax.lax.dot_general(
        hg, wout_ref[...], (((1,), (1,)), ((), ())),
        preferred_element_type=jnp.float32)


@jax.jit
def kernel(x, batch, W_att, W_out):
    n_classes = W_out.shape[0]
    part = _sc_partials(x, batch.astype(jnp.int32), W_att, 0, N)
    return pl.pallas_call(
        _merge_body,
        in_specs=[
            pl.BlockSpec((NW, NUM_GRAPHS, D), lambda: (0, 0, 0)),
            pl.BlockSpec((n_classes, D), lambda: (0, 0)),
        ],
        out_specs=pl.BlockSpec((NUM_GRAPHS, n_classes), lambda: (0, 0)),
        out_shape=jax.ShapeDtypeStruct((NUM_GRAPHS, n_classes), jnp.float32),
    )(part, W_out)


# speedup vs baseline: 3.2685x; 3.2685x over previous
import jax
import jax.numpy as jnp
from jax import lax
from jax.experimental import pallas as pl
from jax.experimental.pallas import tpu as pltpu

import kernel_sc4 as sc

NUM_GRAPHS = 64
D = 512
N = 100000
BLOCK_ROWS = 2000
N_SC = 22000
N_TC = N - N_SC
NW = sc.NW


def _tc_body(lo_ref, hi_ref, x_ref, b_ref, watt_ref, hg_ref):
    i = pl.program_id(0)

    @pl.when(i == 0)
    def _init():
        hg_ref[...] = jnp.full_like(hg_ref, -jnp.inf)

    xb = x_ref[...]
    att = jax.lax.dot_general(
        xb, watt_ref[...], (((1,), (1,)), ((), ())),
        preferred_element_type=jnp.float32)
    scale = (jax.nn.sigmoid(att) + 1.0) * 0.5
    y = xb * scale
    bcol = b_ref[0]

    def seg_body(s, carry):
        m = bcol == s
        col = jnp.max(jnp.where(m, y, -jnp.inf), axis=0, keepdims=True)
        hg_ref[pl.ds(s, 1), :] = jnp.maximum(hg_ref[pl.ds(s, 1), :], col)
        return carry

    jax.lax.fori_loop(lo_ref[i], hi_ref[i] + 1, seg_body, 0)


def _tc_partials(x, batch_r, blk_lo, blk_hi, W_att):
    b = BLOCK_ROWS
    nb = N_TC // b
    grid_spec = pltpu.PrefetchScalarGridSpec(
        num_scalar_prefetch=2,
        grid=(nb,),
        in_specs=[
            pl.BlockSpec((b, D), lambda i, lo, hi: (i, 0)),
            pl.BlockSpec((1, b, 1), lambda i, lo, hi: (i, 0, 0)),
            pl.BlockSpec((1, D), lambda i, lo, hi: (0, 0)),
        ],
        out_specs=pl.BlockSpec((NUM_GRAPHS, D), lambda i, lo, hi: (0, 0)),
    )
    return pl.pallas_call(
        _tc_body,
        grid_spec=grid_spec,
        out_shape=jax.ShapeDtypeStruct((NUM_GRAPHS, D), jnp.float32),
    )(blk_lo, blk_hi, x, batch_r, W_att)


def _merge_body(hg_ref, part_ref, wout_ref, out_ref):
    def body(w, m):
        return jnp.maximum(m, part_ref[w])

    hg = lax.fori_loop(0, NW, body, hg_ref[...])
    out_ref[...] = jax.lax.dot_general(
        hg, wout_ref[...], (((1,), (1,)), ((), ())),
        preferred_element_type=jnp.float32)


@jax.jit
def kernel(x, batch, W_att, W_out):
    n_classes = W_out.shape[0]
    batch = batch.astype(jnp.int32)

    part = sc._sc_partials(x, batch, W_att, N_TC, N_SC)

    nb = N_TC // BLOCK_ROWS
    b_tc = batch[:N_TC]
    batch_r = b_tc.reshape(nb, BLOCK_ROWS, 1)
    blk_lo = b_tc[::BLOCK_ROWS]
    blk_hi = b_tc[BLOCK_ROWS - 1::BLOCK_ROWS]
    hg_tc = _tc_partials(x, batch_r, blk_lo, blk_hi, W_att)

    return pl.pallas_call(
        _merge_body,
        in_specs=[
            pl.BlockSpec((NUM_GRAPHS, D), lambda: (0, 0)),
            pl.BlockSpec((NW, NUM_GRAPHS, D), lambda: (0, 0, 0)),
            pl.BlockSpec((n_classes, D), lambda: (0, 0)),
        ],
        out_specs=pl.BlockSpec((NUM_GRAPHS, n_classes), lambda: (0, 0)),
        out_shape=jax.ShapeDtypeStruct((NUM_GRAPHS, n_classes), jnp.float32),
    )(hg_tc, part, W_out)
